# baseline (speedup 1.0000x reference)
import jax
import jax.numpy as jnp
from jax.experimental import pallas as pl
from jax.experimental.pallas import tpu as pltpu

_B = 32
_P = 8732
_C = 21
_G = 10
_CH = 384
_NSTEP = 23
_PPAD = _CH * _NSTEP
_THRESH = 0.5
_NEGPOS = 3
_V0 = 0.1
_V1 = 0.2


def _fused_body(pb_ref, tg_ref, xt_ref, lt_ref, out_ref,
                bto_s, bti_s, bpv_s, bpi_s, lossc_s, acc_s, npf_s):
    ph = pl.program_id(0)
    j = pl.program_id(1)
    f32 = jnp.float32

    lane = jax.lax.broadcasted_iota(jnp.int32, (_B, _CH), 1) + j * _CH
    valid = lane < _P

    cx = pb_ref[0:1, :]
    cy = pb_ref[1:2, :]
    w = pb_ref[2:3, :]
    h = pb_ref[3:4, :]
    tg = tg_ref[...]

    @pl.when(jnp.logical_and(ph == 0, j == 0))
    def _init0():
        bpv_s[...] = jnp.full((_B, 128), -2.0, f32)
        bpi_s[...] = jnp.zeros((_B, 128), jnp.int32)
        acc_s[...] = jnp.zeros((_B, _CH), f32)
        npf_s[...] = jnp.zeros((_B, _CH), f32)

    @pl.when(ph == 0)
    def _match():
        px1 = cx - w * 0.5
        py1 = cy - h * 0.5
        px2 = cx + w * 0.5
        py2 = cy + h * 0.5
        area_b = (px2 - px1) * (py2 - py1)
        bto = jnp.full((_B, _CH), -1.0, f32)
        bti = jnp.zeros((_B, _CH), jnp.int32)
        for g in range(_G):
            tx1 = tg[0][:, g:g + 1]
            ty1 = tg[1][:, g:g + 1]
            tx2 = tg[2][:, g:g + 1]
            ty2 = tg[3][:, g:g + 1]
            iw = jnp.maximum(jnp.minimum(tx2, px2) - jnp.maximum(tx1, px1), 0.0)
            ih = jnp.maximum(jnp.minimum(ty2, py2) - jnp.maximum(ty1, py1), 0.0)
            inter = iw * ih
            area_a = (tx2 - tx1) * (ty2 - ty1)
            ov = inter / (area_a + area_b - inter + 1e-8)
            upd = ov > bto
            bti = jnp.where(upd, g, bti)
            bto = jnp.where(upd, ov, bto)
            ovm = jnp.where(valid, ov, -1.0)
            cmx = jnp.max(ovm, axis=1, keepdims=True)
            cid = jnp.min(jnp.where(ovm >= cmx, lane, _P),
                          axis=1, keepdims=True)
            pv = bpv_s[:, g:g + 1]
            take = cmx > pv
            bpv_s[:, g:g + 1] = jnp.where(take, cmx, pv)
            bpi_s[:, g:g + 1] = jnp.where(take, cid, bpi_s[:, g:g + 1])
        bto_s[:, pl.ds(j * _CH, _CH)] = bto
        bti_s[:, pl.ds(j * _CH, _CH)] = bti

    @pl.when(ph == 1)
    def _stream():
        bto = bto_s[:, pl.ds(j * _CH, _CH)]
        bti = bti_s[:, pl.ds(j * _CH, _CH)]
        for g in range(_G):
            m = lane == bpi_s[:, g:g + 1]
            bto = jnp.where(m, 2.0, bto)
            bti = jnp.where(m, g, bti)

        conf_f = jnp.zeros((_B, _CH), f32)
        mx1 = jnp.zeros((_B, _CH), f32)
        my1 = jnp.zeros((_B, _CH), f32)
        mx2 = jnp.zeros((_B, _CH), f32)
        my2 = jnp.zeros((_B, _CH), f32)
        for g in range(_G):
            m = bti == g
            conf_f = jnp.where(m, tg[4][:, g:g + 1], conf_f)
            mx1 = jnp.where(m, tg[0][:, g:g + 1], mx1)
            my1 = jnp.where(m, tg[1][:, g:g + 1], my1)
            mx2 = jnp.where(m, tg[2][:, g:g + 1], mx2)
            my2 = jnp.where(m, tg[3][:, g:g + 1], my2)
        conf_f = jnp.where(bto < _THRESH, 0.0, conf_f)
        conf = conf_f.astype(jnp.int32)
        pos = (conf > 0) & valid

        lt0 = ((mx1 + mx2) * 0.5 - cx) / (_V0 * w)
        lt1 = ((my1 + my2) * 0.5 - cy) / (_V0 * h)
        lt2 = jnp.log(jnp.maximum((mx2 - mx1) / w, 1e-8)) * (1.0 / _V1)
        lt3 = jnp.log(jnp.maximum((my2 - my1) / h, 1e-8)) * (1.0 / _V1)

        sl = jnp.zeros((_B, _CH), f32)
        for c, ltc in enumerate((lt0, lt1, lt2, lt3)):
            d = lt_ref[c] - ltc
            a = jnp.abs(d)
            sl = sl + jnp.where(a < 1.0, 0.5 * d * d, a - 0.5)

        sumexp = jnp.zeros((_B, _CH), f32)
        xg = jnp.zeros((_B, _CH), f32)
        for c in range(_C):
            xc = xt_ref[c]
            sumexp = sumexp + jnp.exp(xc)
            xg = jnp.where(conf == c, xc, xg)
        ce = jnp.log(sumexp) - xg
        ce = jnp.where(valid, ce, 0.0)

        lossc_s[:, pl.ds(j * _CH, _CH)] = jnp.where(pos, 0.0, ce)
        acc_s[...] += jnp.where(pos, ce + sl, 0.0)
        npf_s[...] += jnp.where(pos, 1.0, 0.0)

    @pl.when(jnp.logical_and(ph == 1, j == _NSTEP - 1))
    def _select():
        np_b = jnp.sum(npf_s[...], axis=1, keepdims=True)
        k = jnp.minimum(_NEGPOS * np_b, float(_P - 1))
        n_tot = jnp.sum(np_b)
        base = jnp.sum(acc_s[...])

        def body(_, carry):
            lo, hi = carry
            mid = lo + jax.lax.shift_right_logical(hi - lo, 1)
            vb = jax.lax.bitcast_convert_type(lossc_s[...], jnp.int32)
            cnt = jnp.sum(jnp.where(vb >= mid, 1.0, 0.0), axis=1, keepdims=True)
            ge = cnt >= k
            return jnp.where(ge, mid, lo), jnp.where(ge, hi, mid)

        lo0 = jnp.zeros((_B, 1), jnp.int32)
        hi0 = jnp.full((_B, 1), jnp.int32(0x7F800001))
        t_bits, _ = jax.lax.fori_loop(0, 31, body, (lo0, hi0))
        t_val = jax.lax.bitcast_convert_type(t_bits, f32)

        v = lossc_s[...]
        vb = jax.lax.bitcast_convert_type(v, jnp.int32)
        gt = vb > t_bits
        m = jnp.sum(jnp.where(gt, 1.0, 0.0), axis=1, keepdims=True)
        s = jnp.sum(jnp.where(gt, v, 0.0), axis=1, keepdims=True)
        neg = s + (k - m) * t_val
        neg = jnp.where(k >= 1.0, neg, 0.0)

        denom = jnp.maximum(n_tot, 1.0)
        out_ref[...] = ((base + jnp.sum(neg)) / denom).reshape(1, 1)


def _loss(loc_preds, cls_preds, priorbox, targets, interpret=False):
    f32 = jnp.float32
    xt = jnp.transpose(cls_preds, (2, 0, 1))
    lt = jnp.transpose(loc_preds, (2, 0, 1))
    pbt = jnp.transpose(priorbox, (1, 0))
    tgt = jnp.transpose(targets, (2, 0, 1))
    call = pl.pallas_call(
        _fused_body,
        grid=(2, _NSTEP),
        in_specs=[
            pl.BlockSpec((4, _CH), lambda ph, j: (0, j)),
            pl.BlockSpec((5, _B, _G), lambda ph, j: (0, 0, 0)),
            pl.BlockSpec((_C, _B, _CH), lambda ph, j: (0, 0, j * ph)),
            pl.BlockSpec((4, _B, _CH), lambda ph, j: (0, 0, j * ph)),
        ],
        out_specs=pl.BlockSpec((1, 1), lambda ph, j: (0, 0)),
        out_shape=jax.ShapeDtypeStruct((1, 1), f32),
        scratch_shapes=[
            pltpu.VMEM((_B, _PPAD), f32),
            pltpu.VMEM((_B, _PPAD), jnp.int32),
            pltpu.VMEM((_B, 128), f32),
            pltpu.VMEM((_B, 128), jnp.int32),
            pltpu.VMEM((_B, _PPAD), f32),
            pltpu.VMEM((_B, _CH), f32),
            pltpu.VMEM((_B, _CH), f32),
        ],
        interpret=interpret,
    )
    out = call(pbt, tgt, xt, lt)
    return out.reshape(())


def kernel(loc_preds, cls_preds, priorbox, targets):
    return _loss(loc_preds, cls_preds, priorbox, targets)
